# baseline (speedup 1.0000x reference)
import functools

import jax
import jax.numpy as jnp
from jax import lax
from jax.experimental import pallas as pl
from jax.experimental.pallas import tpu as pltpu
from jax.experimental.pallas import tpu_sc as plsc

_NC = 2
_NS = 16
_NW = _NC * _NS
_LANES = 16
_CHUNK = 16
_NBUF = 4


@functools.partial(jax.jit, static_argnums=(1, 2, 3))
def _broadcast_rows(emb_table, batch, max_len, d):
    rows_per_w = max_len // _NW
    n_chunks = rows_per_w // _CHUNK

    def body(table_hbm, out_hbm, *refs):
        bufs = refs[:_NBUF]
        idxs = refs[_NBUF:2 * _NBUF]
        in_sem, out_sem = refs[2 * _NBUF], refs[2 * _NBUF + 1]
        c = lax.axis_index("c")
        s = lax.axis_index("s")
        wid = s * _NC + c
        base = wid * rows_per_w
        lane = lax.iota(jnp.int32, _LANES)

        def in_copy(i):
            row0 = base + i * _CHUNK + 1
            idx = idxs[i % _NBUF]
            for j in range(_CHUNK // _LANES):
                idx[pl.ds(j * _LANES, _LANES)] = row0 + j * _LANES + lane
            return pltpu.make_async_copy(
                table_hbm.at[idx], bufs[i % _NBUF], in_sem)

        def out_copies(i):
            row0 = base + i * _CHUNK
            return [
                pltpu.make_async_copy(
                    bufs[i % _NBUF],
                    out_hbm.at[b, pl.ds(row0, _CHUNK)],
                    out_sem)
                for b in range(batch)
            ]

        in_h = [None] * n_chunks
        out_h = [None] * n_chunks
        in_h[0] = in_copy(0)
        in_h[0].start()
        drained = 0
        for i in range(n_chunks):
            in_h[i].wait()
            out_h[i] = out_copies(i)
            for cp in out_h[i]:
                cp.start()
            if i + 1 < n_chunks:
                if i + 1 >= _NBUF:
                    for cp in out_h[i + 1 - _NBUF]:
                        cp.wait()
                    drained = i + 2 - _NBUF
                in_h[i + 1] = in_copy(i + 1)
                in_h[i + 1].start()
        for i in range(drained, n_chunks):
            for cp in out_h[i]:
                cp.wait()

    return pl.kernel(
        body,
        out_type=jax.ShapeDtypeStruct((batch, max_len, d), emb_table.dtype),
        mesh=plsc.VectorSubcoreMesh(
            core_axis_name="c", subcore_axis_name="s",
            num_cores=_NC, num_subcores=_NS),
        scratch_types=(
            [pltpu.VMEM((_CHUNK, d), emb_table.dtype) for _ in range(_NBUF)]
            + [pltpu.VMEM((_CHUNK,), jnp.int32) for _ in range(_NBUF)]
            + [pltpu.SemaphoreType.DMA, pltpu.SemaphoreType.DMA]
        ),
    )(emb_table)


def kernel(x, emb_table):
    batch, max_len = x.shape
    d = emb_table.shape[1]
    return _broadcast_rows(emb_table, batch, max_len, d)


# speedup vs baseline: 1.0676x; 1.0676x over previous
import functools

import jax
import jax.numpy as jnp
from jax import lax
from jax.experimental import pallas as pl
from jax.experimental.pallas import tpu as pltpu
from jax.experimental.pallas import tpu_sc as plsc

_NC = 2
_NS = 16
_NW = _NC * _NS
_LANES = 16
_CHUNK = 32
_NBUF = 3


@functools.partial(jax.jit, static_argnums=(1, 2, 3))
def _broadcast_rows(emb_table, batch, max_len, d):
    rows_per_w = max_len // _NW
    n_chunks = rows_per_w // _CHUNK

    def body(table_hbm, out_hbm, *refs):
        bufs = refs[:_NBUF]
        idxs = refs[_NBUF:2 * _NBUF]
        in_sem, out_sem = refs[2 * _NBUF], refs[2 * _NBUF + 1]
        c = lax.axis_index("c")
        s = lax.axis_index("s")
        wid = s * _NC + c
        base = wid * rows_per_w
        lane = lax.iota(jnp.int32, _LANES)

        def in_copy(i):
            row0 = base + i * _CHUNK + 1
            idx = idxs[i % _NBUF]
            for j in range(_CHUNK // _LANES):
                idx[pl.ds(j * _LANES, _LANES)] = row0 + j * _LANES + lane
            return pltpu.make_async_copy(
                table_hbm.at[idx], bufs[i % _NBUF], in_sem)

        def out_copies(i):
            row0 = base + i * _CHUNK
            return [
                pltpu.make_async_copy(
                    bufs[i % _NBUF],
                    out_hbm.at[b, pl.ds(row0, _CHUNK)],
                    out_sem)
                for b in range(batch)
            ]

        in_h = [None] * n_chunks
        out_h = [None] * n_chunks
        in_h[0] = in_copy(0)
        in_h[0].start()
        drained = 0
        for i in range(n_chunks):
            in_h[i].wait()
            out_h[i] = out_copies(i)
            for cp in out_h[i]:
                cp.start()
            if i + 1 < n_chunks:
                if i + 1 >= _NBUF:
                    for cp in out_h[i + 1 - _NBUF]:
                        cp.wait()
                    drained = i + 2 - _NBUF
                in_h[i + 1] = in_copy(i + 1)
                in_h[i + 1].start()
        for i in range(drained, n_chunks):
            for cp in out_h[i]:
                cp.wait()

    return pl.kernel(
        body,
        out_type=jax.ShapeDtypeStruct((batch, max_len, d), emb_table.dtype),
        mesh=plsc.VectorSubcoreMesh(
            core_axis_name="c", subcore_axis_name="s",
            num_cores=_NC, num_subcores=_NS),
        scratch_types=(
            [pltpu.VMEM((_CHUNK, d), emb_table.dtype) for _ in range(_NBUF)]
            + [pltpu.VMEM((_CHUNK,), jnp.int32) for _ in range(_NBUF)]
            + [pltpu.SemaphoreType.DMA, pltpu.SemaphoreType.DMA]
        ),
    )(emb_table)


def kernel(x, emb_table):
    batch, max_len = x.shape
    d = emb_table.shape[1]
    return _broadcast_rows(emb_table, batch, max_len, d)


# speedup vs baseline: 1.0815x; 1.0131x over previous
import functools

import jax
import jax.numpy as jnp
from jax import lax
from jax.experimental import pallas as pl
from jax.experimental.pallas import tpu as pltpu
from jax.experimental.pallas import tpu_sc as plsc

_NC = 2
_NS = 16
_NW = _NC * _NS
_LANES = 16
_CHUNKS = (64, 48, 64, 48, 32)
_BUF_ROWS = (64, 48)


@functools.partial(jax.jit, static_argnums=(1, 2, 3))
def _broadcast_rows(emb_table, batch, max_len, d):
    rows_per_w = max_len // _NW
    assert sum(_CHUNKS) == rows_per_w
    n_chunks = len(_CHUNKS)
    starts = [sum(_CHUNKS[:i]) for i in range(n_chunks)]

    def body(table_hbm, out_hbm, buf0, buf1, idx0, idx1, idx_last, in_sem, out_sem):
        c = lax.axis_index("c")
        s = lax.axis_index("s")
        wid = s * _NC + c
        base = wid * rows_per_w
        lane = lax.iota(jnp.int32, _LANES)

        def refs_for(i):
            size = _CHUNKS[i]
            buf = (buf0, buf1)[i % 2]
            if size == _BUF_ROWS[i % 2]:
                idx = (idx0, idx1)[i % 2]
            else:
                idx = idx_last
                buf = buf.at[pl.ds(0, size)]
            return buf, idx, size

        def in_copy(i):
            buf, idx, size = refs_for(i)
            row0 = base + starts[i] + 1
            for j in range(size // _LANES):
                idx[pl.ds(j * _LANES, _LANES)] = row0 + j * _LANES + lane
            return pltpu.make_async_copy(table_hbm.at[idx], buf, in_sem)

        def out_copies(i):
            buf, _, size = refs_for(i)
            row0 = base + starts[i]
            return [
                pltpu.make_async_copy(
                    buf, out_hbm.at[b, pl.ds(row0, size)], out_sem)
                for b in range(batch)
            ]

        in_h = [None] * n_chunks
        out_h = [None] * n_chunks
        in_h[0] = in_copy(0)
        in_h[0].start()
        drained = 0
        for i in range(n_chunks):
            in_h[i].wait()
            out_h[i] = out_copies(i)
            for cp in out_h[i]:
                cp.start()
            if i + 1 < n_chunks:
                if i >= 1:
                    for cp in out_h[i - 1]:
                        cp.wait()
                    drained = i
                in_h[i + 1] = in_copy(i + 1)
                in_h[i + 1].start()
        for i in range(drained, n_chunks):
            for cp in out_h[i]:
                cp.wait()

    return pl.kernel(
        body,
        out_type=jax.ShapeDtypeStruct((batch, max_len, d), emb_table.dtype),
        mesh=plsc.VectorSubcoreMesh(
            core_axis_name="c", subcore_axis_name="s",
            num_cores=_NC, num_subcores=_NS),
        scratch_types=[
            pltpu.VMEM((_BUF_ROWS[0], d), emb_table.dtype),
            pltpu.VMEM((_BUF_ROWS[1], d), emb_table.dtype),
            pltpu.VMEM((_BUF_ROWS[0],), jnp.int32),
            pltpu.VMEM((_BUF_ROWS[1],), jnp.int32),
            pltpu.VMEM((_CHUNKS[-1],), jnp.int32),
            pltpu.SemaphoreType.DMA,
            pltpu.SemaphoreType.DMA,
        ],
    )(emb_table)


def kernel(x, emb_table):
    batch, max_len = x.shape
    d = emb_table.shape[1]
    return _broadcast_rows(emb_table, batch, max_len, d)
